# baseline (speedup 1.0000x reference)
import jax
import jax.numpy as jnp
from jax.experimental import pallas as pl
from jax.experimental.pallas import tpu as pltpu

_B = 4
_HW = 24 * 24
_D = 256
_K = 512
_HP = jax.lax.Precision.HIGHEST


def _exact_norm(z, cg):
    diff = z - cg
    sq = diff * diff

    def half_sum(lo):
        acc = sq[:, lo : lo + 8]
        for j in range(1, 16):
            acc = acc + sq[:, lo + 8 * j : lo + 8 * j + 8]
        a4 = acc[:, 0:4] + acc[:, 4:8]
        a2 = a4[:, 0:2] + a4[:, 2:4]
        return a2[:, 0:1] + a2[:, 1:2]

    return jnp.sqrt(half_sum(0) + half_sum(128))


def _vq_body(z_ref, c_ref, out_ref, ct_ref, bias_ref):
    @pl.when(pl.program_id(0) == 0)
    def _():
        ct = jnp.transpose(c_ref[...])
        ct_ref[...] = ct
        bias_ref[...] = 0.5 * jnp.sum(ct * ct, axis=0, keepdims=True)

    z = z_ref[...]
    scores = jax.lax.dot_general(
        z, ct_ref[...], (((1,), (0,)), ((), ())),
        precision=_HP, preferred_element_type=jnp.float32)
    d = bias_ref[...] - scores
    kiota = jax.lax.broadcasted_iota(jnp.int32, d.shape, 1)

    m1 = jnp.min(d, axis=1, keepdims=True)
    i1 = jnp.min(jnp.where(d == m1, kiota, _K), axis=1, keepdims=True)
    d2 = jnp.where(kiota == i1, jnp.inf, d)
    m2 = jnp.min(d2, axis=1, keepdims=True)
    i2 = jnp.min(jnp.where(d2 == m2, kiota, _K), axis=1, keepdims=True)

    c = c_ref[...]
    oh1 = jnp.where(kiota == i1, 1.0, 0.0)
    oh2 = jnp.where(kiota == i2, 1.0, 0.0)
    cg1 = jax.lax.dot_general(oh1, c, (((1,), (0,)), ((), ())),
                              precision=_HP, preferred_element_type=jnp.float32)
    cg2 = jax.lax.dot_general(oh2, c, (((1,), (0,)), ((), ())),
                              precision=_HP, preferred_element_type=jnp.float32)

    n1 = _exact_norm(z, cg1)
    n2 = _exact_norm(z, cg2)

    lo = jnp.minimum(i1, i2)
    idx = jnp.where(n2 < n1, i2, jnp.where(n1 < n2, i1, lo))
    out_ref[...] = idx[:, 0].reshape(1, 24, 24)


def kernel(z_e, codebook):
    z2d = z_e.reshape(_B * _HW, _D)
    out = pl.pallas_call(
        _vq_body,
        grid=(_B,),
        in_specs=[
            pl.BlockSpec((_HW, _D), lambda b: (b, 0)),
            pl.BlockSpec((_K, _D), lambda b: (0, 0)),
        ],
        out_specs=pl.BlockSpec((1, 24, 24), lambda b: (b, 0, 0)),
        out_shape=jax.ShapeDtypeStruct((_B, 24, 24), jnp.int32),
        scratch_shapes=[
            pltpu.VMEM((_D, _K), jnp.float32),
            pltpu.VMEM((1, _K), jnp.float32),
        ],
    )(z2d, codebook)
    return out


# speedup vs baseline: 1.7962x; 1.7962x over previous
import jax
import jax.numpy as jnp
from jax.experimental import pallas as pl
from jax.experimental.pallas import tpu as pltpu

_B = 4
_HW = 24 * 24
_D = 256
_K = 512
_HP = jax.lax.Precision.HIGHEST


def _vq_body(z_ref, c_ref, out_ref, ct_ref, bias_ref):
    @pl.when(pl.program_id(0) == 0)
    def _():
        ct = jnp.transpose(c_ref[...])
        ct_ref[...] = ct
        bias_ref[...] = 0.5 * jnp.sum(ct * ct, axis=0, keepdims=True)

    z = z_ref[...]
    scores = jax.lax.dot_general(
        z, ct_ref[...], (((1,), (0,)), ((), ())),
        precision=_HP, preferred_element_type=jnp.float32)
    d = bias_ref[...] - scores
    kiota = jax.lax.broadcasted_iota(jnp.int32, d.shape, 1)

    m1 = jnp.min(d, axis=1, keepdims=True)
    i1 = jnp.min(jnp.where(d == m1, kiota, _K), axis=1, keepdims=True)
    d2 = jnp.where(kiota == i1, jnp.inf, d)
    m2 = jnp.min(d2, axis=1, keepdims=True)
    i2 = jnp.min(jnp.where(d2 == m2, kiota, _K), axis=1, keepdims=True)

    i1t = jnp.transpose(i1)
    i2t = jnp.transpose(i2)
    icat = jnp.concatenate([i1t, i2t], axis=1)

    kio2 = jax.lax.broadcasted_iota(jnp.int32, (_K, 2 * _HW), 0)
    oht = jnp.where(kio2 == icat, 1.0, 0.0)
    cgt = jax.lax.dot_general(
        ct_ref[...], oht, (((1,), (0,)), ((), ())),
        precision=_HP, preferred_element_type=jnp.float32)

    zt = jnp.transpose(z)
    ztc = jnp.concatenate([zt, zt], axis=1)
    diff = ztc - cgt
    sq = diff * diff

    def half_sum(lo):
        acc = sq[lo : lo + 8, :]
        for j in range(1, 16):
            acc = acc + sq[lo + 8 * j : lo + 8 * j + 8, :]
        a4 = acc[0:4, :] + acc[4:8, :]
        a2 = a4[0:2, :] + a4[2:4, :]
        return a2[0:1, :] + a2[1:2, :]

    nrm = jnp.sqrt(half_sum(0) + half_sum(128))
    n1 = nrm[:, :_HW]
    n2 = nrm[:, _HW:]

    lo_idx = jnp.minimum(i1t, i2t)
    idx = jnp.where(n2 < n1, i2t, jnp.where(n1 < n2, i1t, lo_idx))
    out_ref[...] = idx.reshape(1, 24, 24)


def kernel(z_e, codebook):
    z2d = z_e.reshape(_B * _HW, _D)
    out = pl.pallas_call(
        _vq_body,
        grid=(_B,),
        in_specs=[
            pl.BlockSpec((_HW, _D), lambda b: (b, 0)),
            pl.BlockSpec((_K, _D), lambda b: (0, 0)),
        ],
        out_specs=pl.BlockSpec((1, 24, 24), lambda b: (b, 0, 0)),
        out_shape=jax.ShapeDtypeStruct((_B, 24, 24), jnp.int32),
        scratch_shapes=[
            pltpu.VMEM((_D, _K), jnp.float32),
            pltpu.VMEM((1, _K), jnp.float32),
        ],
    )(z2d, codebook)
    return out
